# Initial kernel scaffold; baseline (speedup 1.0000x reference)
import functools
import math

import jax
import jax.numpy as jnp
from jax import lax
from jax.experimental import pallas as pl
from jax.experimental.pallas import tpu as pltpu
from jax.experimental.pallas import tpu_sc as plsc

N = 10000
NP = 10240
E = 320000
CH = 128
NCH = E // CH
NC = 2
NS = 16
NW = NC * NS
ROWS_W = NP // NS
BASE_TRIPS = NCH // NW
EXTRA = NCH - BASE_TRIPS * NW

BN_SCALE = 1.0 / math.sqrt(1.0 + 1e-5)

_MESH = plsc.VectorSubcoreMesh(core_axis_name="c", subcore_axis_name="s")


@functools.partial(
    pl.kernel,
    mesh=_MESH,
    out_type=jax.ShapeDtypeStruct((2 * NP,), jnp.float32),
    scratch_types=[
        pltpu.VMEM((CH,), jnp.int32),
        pltpu.VMEM((CH,), jnp.float32),
        pltpu.VMEM_SHARED((NP,), jnp.float32),
    ],
)
def _deg_sc(dst_hbm, zeros_hbm, ones_hbm, out_hbm, idx_v, ones_v, acc_sh):
    c = lax.axis_index("c")
    s = lax.axis_index("s")
    w = s * NC + c
    pltpu.sync_copy(zeros_hbm.at[pl.ds(s * ROWS_W, ROWS_W)],
                    acc_sh.at[pl.ds(s * ROWS_W, ROWS_W)])
    pltpu.sync_copy(ones_hbm, ones_v)
    plsc.subcore_barrier()

    def body(j, carry):
        base = (w + j * NW) * CH
        pltpu.sync_copy(dst_hbm.at[pl.ds(base, CH)], idx_v)
        pltpu.sync_copy(ones_v, acc_sh.at[idx_v], add=True)
        return carry

    trips = BASE_TRIPS + jnp.where(w < EXTRA, 1, 0)
    lax.fori_loop(0, trips, body, 0)
    plsc.subcore_barrier()
    pltpu.sync_copy(acc_sh.at[pl.ds(s * ROWS_W, ROWS_W)],
                    out_hbm.at[pl.ds(c * NP + s * ROWS_W, ROWS_W)])


def _make_agg(D):
    @functools.partial(
        pl.kernel,
        mesh=_MESH,
        out_type=jax.ShapeDtypeStruct((2 * NP, D), jnp.float32),
        scratch_types=[
            pltpu.VMEM((CH,), jnp.int32),
            pltpu.VMEM((CH,), jnp.int32),
            pltpu.VMEM((CH, D), jnp.float32),
            pltpu.VMEM_SHARED((NP, D), jnp.float32),
            pltpu.SemaphoreType.DMA,
        ],
    )
    def agg(hs_hbm, src_hbm, dst_hbm, zeros_hbm, out_hbm,
            sidx_v, didx_v, buf_v, acc_sh, sem):
        c = lax.axis_index("c")
        s = lax.axis_index("s")
        w = s * NC + c
        pltpu.sync_copy(zeros_hbm.at[pl.ds(s * ROWS_W, ROWS_W)],
                        acc_sh.at[pl.ds(s * ROWS_W, ROWS_W)])
        plsc.subcore_barrier()

        def body(j, carry):
            base = (w + j * NW) * CH
            pltpu.sync_copy(src_hbm.at[pl.ds(base, CH)], sidx_v)
            pltpu.sync_copy(dst_hbm.at[pl.ds(base, CH)], didx_v)
            pltpu.async_copy(hs_hbm.at[sidx_v], buf_v, sem).wait()
            pltpu.sync_copy(buf_v, acc_sh.at[didx_v], add=True)
            return carry

        trips = BASE_TRIPS + jnp.where(w < EXTRA, 1, 0)
        lax.fori_loop(0, trips, body, 0)
        plsc.subcore_barrier()
        pltpu.sync_copy(acc_sh.at[pl.ds(s * ROWS_W, ROWS_W)],
                        out_hbm.at[pl.ds(c * NP + s * ROWS_W, ROWS_W)])

    return agg


_agg128 = _make_agg(128)
_agg64 = _make_agg(64)


_BLK = 640
_GRID = NP // _BLK


def _row_spec(d):
    return pl.BlockSpec((_BLK, d), lambda i: (i, 0))


def _full_spec(r, d):
    return pl.BlockSpec((r, d), lambda i: (0, 0))


def _t0_body(d0_ref, d1_ref, x_ref, w_ref, hs_ref, dinv_ref):
    deg = d0_ref[...] + d1_ref[...] + 1.0
    dv = jnp.where(deg > 0, lax.rsqrt(deg), 0.0)
    h = jnp.dot(x_ref[...], w_ref[...], preferred_element_type=jnp.float32)
    hs_ref[...] = h * dv
    dinv_ref[...] = dv


def _t0(d0, d1, xp, W1):
    return pl.pallas_call(
        _t0_body,
        grid=(_GRID,),
        in_specs=[_row_spec(1), _row_spec(1), _row_spec(128), _full_spec(128, 128)],
        out_specs=[_row_spec(128), _row_spec(1)],
        out_shape=[jax.ShapeDtypeStruct((NP, 128), jnp.float32),
                   jax.ShapeDtypeStruct((NP, 1), jnp.float32)],
    )(d0, d1, xp, W1)


def _t1_body(y0_ref, y1_ref, hs_ref, dinv_ref, b_ref, g_ref, be_ref, w_ref, out_ref):
    dv = dinv_ref[...]
    h = dv * (y0_ref[...] + y1_ref[...] + hs_ref[...]) + b_ref[...]
    t = h * (g_ref[...] * BN_SCALE) + be_ref[...]
    r = jnp.maximum(t, 0.0)
    out_ref[...] = dv * jnp.dot(r, w_ref[...], preferred_element_type=jnp.float32)


def _t1(y0, y1, hs, dinv, b, g, be, W, dout):
    return pl.pallas_call(
        _t1_body,
        grid=(_GRID,),
        in_specs=[_row_spec(128), _row_spec(128), _row_spec(128), _row_spec(1),
                  _full_spec(1, 128), _full_spec(1, 128), _full_spec(1, 128),
                  _full_spec(128, dout)],
        out_specs=_row_spec(dout),
        out_shape=jax.ShapeDtypeStruct((NP, dout), jnp.float32),
    )(y0, y1, hs, dinv, b, g, be, W)


def _t2_body(y0_ref, y1_ref, hs_ref, dinv_ref, b3_ref, wv_ref, bv_ref,
             ow_ref, ob_ref, out_ref):
    dv = dinv_ref[...]
    h3 = dv * (y0_ref[...] + y1_ref[...] + hs_ref[...]) + b3_ref[...]
    t = lax.dot_general(h3, wv_ref[...], (((1,), (1,)), ((), ())),
                        preferred_element_type=jnp.float32) + bv_ref[...]
    a = lax.dot_general(t, ow_ref[...], (((1,), (1,)), ((), ())),
                        preferred_element_type=jnp.float32) + ob_ref[...]
    out_ref[...] = h3 + a


def _t2(y0, y1, hs, dinv, b3, wv, bv, ow, ob):
    return pl.pallas_call(
        _t2_body,
        grid=(_GRID,),
        in_specs=[_row_spec(64), _row_spec(64), _row_spec(64), _row_spec(1),
                  _full_spec(1, 64), _full_spec(64, 64), _full_spec(1, 64),
                  _full_spec(64, 64), _full_spec(1, 64)],
        out_specs=_row_spec(64),
        out_shape=jax.ShapeDtypeStruct((NP, 64), jnp.float32),
    )(y0, y1, hs, dinv, b3, wv, bv, ow, ob)


def kernel(x, edge_index, W1, b1, g1, be1, W2, b2, g2, be2, W3, b3,
           in_w, in_b, out_w, out_b):
    src = edge_index[0].astype(jnp.int32)
    dst = edge_index[1].astype(jnp.int32)
    xp = jnp.pad(x, ((0, NP - N), (0, 0)))

    zeros1 = jnp.zeros((NP,), jnp.float32)
    zeros128 = jnp.zeros((NP, 128), jnp.float32)
    zeros64 = jnp.zeros((NP, 64), jnp.float32)
    ones = jnp.ones((CH,), jnp.float32)

    degpair = _deg_sc(dst, zeros1, ones)
    d0 = degpair[:NP].reshape(NP, 1)
    d1 = degpair[NP:].reshape(NP, 1)

    hs1, dinv = _t0(d0, d1, xp, W1)

    y = _agg128(hs1, src, dst, zeros128)
    hs2 = _t1(y[:NP], y[NP:], hs1, dinv,
              b1.reshape(1, 128), g1.reshape(1, 128), be1.reshape(1, 128),
              W2, 128)

    y = _agg128(hs2, src, dst, zeros128)
    hs3 = _t1(y[:NP], y[NP:], hs2, dinv,
              b2.reshape(1, 128), g2.reshape(1, 128), be2.reshape(1, 128),
              W3, 64)

    y3 = _agg64(hs3, src, dst, zeros64)
    out = _t2(y3[:NP], y3[NP:], hs3, dinv,
              b3.reshape(1, 64), in_w[128:192], in_b[128:192].reshape(1, 64),
              out_w, out_b.reshape(1, 64))
    return out[:N]


# speedup vs baseline: 13.0804x; 13.0804x over previous
import functools
import math

import jax
import jax.numpy as jnp
from jax import lax
from jax.experimental import pallas as pl
from jax.experimental.pallas import tpu as pltpu
from jax.experimental.pallas import tpu_sc as plsc

N = 10000
NP = 10240
E = 320000
CH = 128
NCH = E // CH
NC = 2
NS = 16
NW = NC * NS
ROWS_W = NP // NS
BASE_TRIPS = NCH // NW
EXTRA = NCH - BASE_TRIPS * NW

BN_SCALE = 1.0 / math.sqrt(1.0 + 1e-5)

_MESH = plsc.VectorSubcoreMesh(core_axis_name="c", subcore_axis_name="s")


@functools.partial(
    pl.kernel,
    mesh=_MESH,
    out_type=jax.ShapeDtypeStruct((2 * NP,), jnp.float32),
    scratch_types=[
        pltpu.VMEM((CH,), jnp.int32),
        pltpu.VMEM((CH,), jnp.float32),
        pltpu.VMEM_SHARED((NP,), jnp.float32),
    ],
)
def _deg_sc(dst_hbm, zeros_hbm, ones_hbm, out_hbm, idx_v, ones_v, acc_sh):
    c = lax.axis_index("c")
    s = lax.axis_index("s")
    w = s * NC + c
    pltpu.sync_copy(zeros_hbm.at[pl.ds(s * ROWS_W, ROWS_W)],
                    acc_sh.at[pl.ds(s * ROWS_W, ROWS_W)])
    pltpu.sync_copy(ones_hbm, ones_v)
    plsc.subcore_barrier()

    def body(j, carry):
        base = (w + j * NW) * CH
        pltpu.sync_copy(dst_hbm.at[pl.ds(base, CH)], idx_v)
        pltpu.sync_copy(ones_v, acc_sh.at[idx_v], add=True)
        return carry

    trips = BASE_TRIPS + jnp.where(w < EXTRA, 1, 0)
    lax.fori_loop(0, trips, body, 0)
    plsc.subcore_barrier()
    pltpu.sync_copy(acc_sh.at[pl.ds(s * ROWS_W, ROWS_W)],
                    out_hbm.at[pl.ds(c * NP + s * ROWS_W, ROWS_W)])


def _make_agg(D):
    @functools.partial(
        pl.kernel,
        mesh=_MESH,
        out_type=jax.ShapeDtypeStruct((2 * NP, D), jnp.float32),
        scratch_types=[
            pltpu.VMEM((CH,), jnp.int32),
            pltpu.VMEM((CH,), jnp.int32),
            pltpu.VMEM((CH, D), jnp.float32),
            pltpu.VMEM_SHARED((NP, D), jnp.float32),
            pltpu.SemaphoreType.DMA,
        ],
    )
    def agg(hs_hbm, src_hbm, dst_hbm, zeros_hbm, out_hbm,
            sidx_v, didx_v, buf_v, acc_sh, sem):
        c = lax.axis_index("c")
        s = lax.axis_index("s")
        w = s * NC + c
        pltpu.sync_copy(zeros_hbm.at[pl.ds(s * ROWS_W, ROWS_W)],
                        acc_sh.at[pl.ds(s * ROWS_W, ROWS_W)])
        plsc.subcore_barrier()

        def body(j, carry):
            base = (w + j * NW) * CH
            pltpu.sync_copy(src_hbm.at[pl.ds(base, CH)], sidx_v)
            pltpu.sync_copy(dst_hbm.at[pl.ds(base, CH)], didx_v)
            pltpu.async_copy(hs_hbm.at[sidx_v], buf_v, sem).wait()
            pltpu.sync_copy(buf_v, acc_sh.at[didx_v], add=True)
            return carry

        trips = BASE_TRIPS + jnp.where(w < EXTRA, 1, 0)
        lax.fori_loop(0, trips, body, 0)
        plsc.subcore_barrier()
        pltpu.sync_copy(acc_sh.at[pl.ds(s * ROWS_W, ROWS_W)],
                        out_hbm.at[pl.ds(c * NP + s * ROWS_W, ROWS_W)])

    return agg


_agg128 = _make_agg(128)


_BLK = 640
_GRID = NP // _BLK


def _row_spec(d):
    return pl.BlockSpec((_BLK, d), lambda i: (i, 0))


def _full_spec(r, d):
    return pl.BlockSpec((r, d), lambda i: (0, 0))


def _t0_body(d0_ref, d1_ref, x_ref, w_ref, hs_ref, dinv_ref):
    deg = d0_ref[...] + d1_ref[...] + 1.0
    dv = jnp.where(deg > 0, lax.rsqrt(deg), 0.0)
    h = jnp.dot(x_ref[...], w_ref[...], preferred_element_type=jnp.float32)
    hs_ref[...] = h * dv
    dinv_ref[...] = dv


def _t0(d0, d1, xp, W1):
    return pl.pallas_call(
        _t0_body,
        grid=(_GRID,),
        in_specs=[_row_spec(1), _row_spec(1), _row_spec(128), _full_spec(128, 128)],
        out_specs=[_row_spec(128), _row_spec(1)],
        out_shape=[jax.ShapeDtypeStruct((NP, 128), jnp.float32),
                   jax.ShapeDtypeStruct((NP, 1), jnp.float32)],
    )(d0, d1, xp, W1)


def _t1_body(y0_ref, y1_ref, hs_ref, dinv_ref, b_ref, g_ref, be_ref, w_ref, out_ref):
    dv = dinv_ref[...]
    h = dv * (y0_ref[...] + y1_ref[...] + hs_ref[...]) + b_ref[...]
    t = h * (g_ref[...] * BN_SCALE) + be_ref[...]
    r = jnp.maximum(t, 0.0)
    out_ref[...] = dv * jnp.dot(r, w_ref[...], preferred_element_type=jnp.float32)


def _t1(y0, y1, hs, dinv, b, g, be, W):
    return pl.pallas_call(
        _t1_body,
        grid=(_GRID,),
        in_specs=[_row_spec(128), _row_spec(128), _row_spec(128), _row_spec(1),
                  _full_spec(1, 128), _full_spec(1, 128), _full_spec(1, 128),
                  _full_spec(128, 128)],
        out_specs=_row_spec(128),
        out_shape=jax.ShapeDtypeStruct((NP, 128), jnp.float32),
    )(y0, y1, hs, dinv, b, g, be, W)


def _t1b_body(y0_ref, y1_ref, hs_ref, dinv_ref, b_ref, g_ref, be_ref, out_ref):
    dv = dinv_ref[...]
    h = dv * (y0_ref[...] + y1_ref[...] + hs_ref[...]) + b_ref[...]
    t = h * (g_ref[...] * BN_SCALE) + be_ref[...]
    out_ref[...] = dv * jnp.maximum(t, 0.0)


def _t1b(y0, y1, hs, dinv, b, g, be):
    return pl.pallas_call(
        _t1b_body,
        grid=(_GRID,),
        in_specs=[_row_spec(128), _row_spec(128), _row_spec(128), _row_spec(1),
                  _full_spec(1, 128), _full_spec(1, 128), _full_spec(1, 128)],
        out_specs=_row_spec(128),
        out_shape=jax.ShapeDtypeStruct((NP, 128), jnp.float32),
    )(y0, y1, hs, dinv, b, g, be)


def _t2_body(y0_ref, y1_ref, hs_ref, dinv_ref, w3_ref, b3_ref, wv_ref, bv_ref,
             ow_ref, ob_ref, out_ref):
    dv = dinv_ref[...]
    z = dv * (y0_ref[...] + y1_ref[...] + hs_ref[...])
    h3 = jnp.dot(z, w3_ref[...], preferred_element_type=jnp.float32) + b3_ref[...]
    t = lax.dot_general(h3, wv_ref[...], (((1,), (1,)), ((), ())),
                        preferred_element_type=jnp.float32) + bv_ref[...]
    a = lax.dot_general(t, ow_ref[...], (((1,), (1,)), ((), ())),
                        preferred_element_type=jnp.float32) + ob_ref[...]
    out_ref[...] = h3 + a


def _t2(y0, y1, hs, dinv, w3, b3, wv, bv, ow, ob):
    return pl.pallas_call(
        _t2_body,
        grid=(_GRID,),
        in_specs=[_row_spec(128), _row_spec(128), _row_spec(128), _row_spec(1),
                  _full_spec(128, 64), _full_spec(1, 64), _full_spec(64, 64),
                  _full_spec(1, 64), _full_spec(64, 64), _full_spec(1, 64)],
        out_specs=_row_spec(64),
        out_shape=jax.ShapeDtypeStruct((NP, 64), jnp.float32),
    )(y0, y1, hs, dinv, w3, b3, wv, bv, ow, ob)


def kernel(x, edge_index, W1, b1, g1, be1, W2, b2, g2, be2, W3, b3,
           in_w, in_b, out_w, out_b):
    src = edge_index[0].astype(jnp.int32)
    dst = edge_index[1].astype(jnp.int32)
    xp = jnp.pad(x, ((0, NP - N), (0, 0)))

    zeros1 = jnp.zeros((NP,), jnp.float32)
    zeros128 = jnp.zeros((NP, 128), jnp.float32)
    ones = jnp.ones((CH,), jnp.float32)

    degpair = _deg_sc(dst, zeros1, ones)
    d0 = degpair[:NP].reshape(NP, 1)
    d1 = degpair[NP:].reshape(NP, 1)

    hs1, dinv = _t0(d0, d1, xp, W1)

    y = _agg128(hs1, src, dst, zeros128)
    hs2 = _t1(y[:NP], y[NP:], hs1, dinv,
              b1.reshape(1, 128), g1.reshape(1, 128), be1.reshape(1, 128), W2)

    y = _agg128(hs2, src, dst, zeros128)
    hs3 = _t1b(y[:NP], y[NP:], hs2, dinv,
               b2.reshape(1, 128), g2.reshape(1, 128), be2.reshape(1, 128))

    y3 = _agg128(hs3, src, dst, zeros128)
    out = _t2(y3[:NP], y3[NP:], hs3, dinv, W3,
              b3.reshape(1, 64), in_w[128:192], in_b[128:192].reshape(1, 64),
              out_w, out_b.reshape(1, 64))
    return out[:N]


# speedup vs baseline: 26.0706x; 1.9931x over previous
import functools
import math

import jax
import jax.numpy as jnp
from jax import lax
from jax.experimental import pallas as pl
from jax.experimental.pallas import tpu as pltpu
from jax.experimental.pallas import tpu_sc as plsc

N = 10000
NP = 10240
E = 320000
CH = 128
NC = 2
NS = 16
NW = NC * NS
ROWS_W = NP // NS
EPCH = 2560
EP = EPCH * CH
TRIPS = EPCH // NW
HALF = TRIPS // 2

BN_SCALE = 1.0 / math.sqrt(1.0 + 1e-5)

_MESH = plsc.VectorSubcoreMesh(core_axis_name="c", subcore_axis_name="s")


@functools.partial(
    pl.kernel,
    mesh=_MESH,
    out_type=jax.ShapeDtypeStruct((2 * NP,), jnp.float32),
    scratch_types=[
        pltpu.VMEM((TRIPS, CH), jnp.int32),
        pltpu.VMEM((CH,), jnp.float32),
        pltpu.VMEM_SHARED((NP,), jnp.float32),
    ],
)
def _deg_sc(dst_hbm, zeros_hbm, ones_hbm, out_hbm, didx, ones_v, acc_sh):
    c = lax.axis_index("c")
    s = lax.axis_index("s")
    w = s * NC + c
    pltpu.sync_copy(zeros_hbm.at[pl.ds(s * ROWS_W, ROWS_W)],
                    acc_sh.at[pl.ds(s * ROWS_W, ROWS_W)])
    pltpu.sync_copy(ones_hbm, ones_v)
    pltpu.sync_copy(dst_hbm.at[pl.ds(w * TRIPS, TRIPS)], didx)
    plsc.subcore_barrier()

    def body(j, carry):
        pltpu.sync_copy(ones_v, acc_sh.at[didx.at[j]], add=True)
        return carry

    lax.fori_loop(0, TRIPS, body, 0)
    plsc.subcore_barrier()
    pltpu.sync_copy(acc_sh.at[pl.ds(s * ROWS_W, ROWS_W)],
                    out_hbm.at[pl.ds(c * NP + s * ROWS_W, ROWS_W)])


def _make_agg(D):
    @functools.partial(
        pl.kernel,
        mesh=_MESH,
        out_type=jax.ShapeDtypeStruct((2 * NP, D), jnp.float32),
        scratch_types=[
            pltpu.VMEM((HALF, CH), jnp.int32),
            pltpu.VMEM((HALF, CH), jnp.int32),
            pltpu.VMEM((CH, D), jnp.float32),
            pltpu.VMEM((CH, D), jnp.float32),
            pltpu.VMEM_SHARED((NP, D), jnp.float32),
            pltpu.SemaphoreType.DMA,
            pltpu.SemaphoreType.DMA,
        ],
    )
    def agg(hs_hbm, src_hbm, dst_hbm, zeros_hbm, out_hbm,
            sidx, didx, buf0, buf1, acc_sh, gsem0, gsem1):
        c = lax.axis_index("c")
        s = lax.axis_index("s")
        w = s * NC + c
        pltpu.sync_copy(zeros_hbm.at[pl.ds(s * ROWS_W, ROWS_W)],
                        acc_sh.at[pl.ds(s * ROWS_W, ROWS_W)])
        plsc.subcore_barrier()

        for h in range(2):
            pltpu.sync_copy(src_hbm.at[pl.ds(w * TRIPS + h * HALF, HALF)], sidx)
            pltpu.sync_copy(dst_hbm.at[pl.ds(w * TRIPS + h * HALF, HALF)], didx)
            pltpu.async_copy(hs_hbm.at[sidx.at[0]], buf0, gsem0)

            def body(i, carry):
                c0 = 2 * i
                c1 = 2 * i + 1
                d1 = pltpu.async_copy(hs_hbm.at[sidx.at[c1]], buf1, gsem1)
                pltpu.make_async_copy(hs_hbm.at[sidx.at[c0]], buf0, gsem0).wait()
                pltpu.sync_copy(buf0, acc_sh.at[didx.at[c0]], add=True)

                @pl.when(i < HALF // 2 - 1)
                def _():
                    pltpu.async_copy(hs_hbm.at[sidx.at[c0 + 2]], buf0, gsem0)

                d1.wait()
                pltpu.sync_copy(buf1, acc_sh.at[didx.at[c1]], add=True)
                return carry

            lax.fori_loop(0, HALF // 2, body, 0)
        plsc.subcore_barrier()
        pltpu.sync_copy(acc_sh.at[pl.ds(s * ROWS_W, ROWS_W)],
                        out_hbm.at[pl.ds(c * NP + s * ROWS_W, ROWS_W)])

    return agg


_agg128 = _make_agg(128)


_BLK = 640
_GRID = NP // _BLK


def _row_spec(d):
    return pl.BlockSpec((_BLK, d), lambda i: (i, 0))


def _hi_spec(d):
    return pl.BlockSpec((_BLK, d), lambda i: (i + _GRID, 0))


def _full_spec(r, d):
    return pl.BlockSpec((r, d), lambda i: (0, 0))


def _t0_body(d0_ref, d1_ref, x_ref, w_ref, hs_ref, dinv_ref):
    deg = d0_ref[...] + d1_ref[...] + 1.0
    dv = jnp.where(deg > 0, lax.rsqrt(deg), 0.0)
    h = jnp.dot(x_ref[...], w_ref[...], preferred_element_type=jnp.float32)
    hs_ref[...] = h * dv
    dinv_ref[...] = dv


def _t0(degpair, xp, W1):
    return pl.pallas_call(
        _t0_body,
        grid=(_GRID,),
        in_specs=[_row_spec(1), _hi_spec(1), _row_spec(128), _full_spec(128, 128)],
        out_specs=[_row_spec(128), _row_spec(1)],
        out_shape=[jax.ShapeDtypeStruct((NP, 128), jnp.float32),
                   jax.ShapeDtypeStruct((NP, 1), jnp.float32)],
    )(degpair, degpair, xp, W1)


def _t1_body(y0_ref, y1_ref, hs_ref, dinv_ref, b_ref, g_ref, be_ref, w_ref, out_ref):
    dv = dinv_ref[...]
    h = dv * (y0_ref[...] + y1_ref[...] + hs_ref[...]) + b_ref[...]
    t = h * (g_ref[...] * BN_SCALE) + be_ref[...]
    r = jnp.maximum(t, 0.0)
    out_ref[...] = dv * jnp.dot(r, w_ref[...], preferred_element_type=jnp.float32)


def _t1(y, hs, dinv, b, g, be, W):
    return pl.pallas_call(
        _t1_body,
        grid=(_GRID,),
        in_specs=[_row_spec(128), _hi_spec(128), _row_spec(128), _row_spec(1),
                  _full_spec(1, 128), _full_spec(1, 128), _full_spec(1, 128),
                  _full_spec(128, 128)],
        out_specs=_row_spec(128),
        out_shape=jax.ShapeDtypeStruct((NP, 128), jnp.float32),
    )(y, y, hs, dinv, b, g, be, W)


def _t1b_body(y0_ref, y1_ref, hs_ref, dinv_ref, b_ref, g_ref, be_ref, out_ref):
    dv = dinv_ref[...]
    h = dv * (y0_ref[...] + y1_ref[...] + hs_ref[...]) + b_ref[...]
    t = h * (g_ref[...] * BN_SCALE) + be_ref[...]
    out_ref[...] = dv * jnp.maximum(t, 0.0)


def _t1b(y, hs, dinv, b, g, be):
    return pl.pallas_call(
        _t1b_body,
        grid=(_GRID,),
        in_specs=[_row_spec(128), _hi_spec(128), _row_spec(128), _row_spec(1),
                  _full_spec(1, 128), _full_spec(1, 128), _full_spec(1, 128)],
        out_specs=_row_spec(128),
        out_shape=jax.ShapeDtypeStruct((NP, 128), jnp.float32),
    )(y, y, hs, dinv, b, g, be)


def _t2_body(y0_ref, y1_ref, hs_ref, dinv_ref, w3_ref, b3_ref, wv_ref, bv_ref,
             ow_ref, ob_ref, out_ref):
    dv = dinv_ref[...]
    z = dv * (y0_ref[...] + y1_ref[...] + hs_ref[...])
    h3 = jnp.dot(z, w3_ref[...], preferred_element_type=jnp.float32) + b3_ref[...]
    t = lax.dot_general(h3, wv_ref[...], (((1,), (1,)), ((), ())),
                        preferred_element_type=jnp.float32) + bv_ref[...]
    a = lax.dot_general(t, ow_ref[...], (((1,), (1,)), ((), ())),
                        preferred_element_type=jnp.float32) + ob_ref[...]
    out_ref[...] = h3 + a


def _t2(y, hs, dinv, w3, b3, wv, bv, ow, ob):
    return pl.pallas_call(
        _t2_body,
        grid=(_GRID,),
        in_specs=[_row_spec(128), _hi_spec(128), _row_spec(128), _row_spec(1),
                  _full_spec(128, 64), _full_spec(1, 64), _full_spec(64, 64),
                  _full_spec(1, 64), _full_spec(64, 64), _full_spec(1, 64)],
        out_specs=_row_spec(64),
        out_shape=jax.ShapeDtypeStruct((NP, 64), jnp.float32),
    )(y, y, hs, dinv, w3, b3, wv, bv, ow, ob)


def kernel(x, edge_index, W1, b1, g1, be1, W2, b2, g2, be2, W3, b3,
           in_w, in_b, out_w, out_b):
    pad_idx = (N + jnp.arange(EP - E, dtype=jnp.int32) % (NP - N))
    src = jnp.concatenate(
        [edge_index[0].astype(jnp.int32), pad_idx]).reshape(EPCH, CH)
    dst = jnp.concatenate(
        [edge_index[1].astype(jnp.int32), pad_idx]).reshape(EPCH, CH)
    xp = jnp.pad(x, ((0, NP - N), (0, 0)))

    zeros1 = jnp.zeros((NP,), jnp.float32)
    zeros128 = jnp.zeros((NP, 128), jnp.float32)
    ones = jnp.ones((CH,), jnp.float32)

    degpair = _deg_sc(dst, zeros1, ones).reshape(2 * NP, 1)

    hs1, dinv = _t0(degpair, xp, W1)

    y = _agg128(hs1, src, dst, zeros128)
    hs2 = _t1(y, hs1, dinv,
              b1.reshape(1, 128), g1.reshape(1, 128), be1.reshape(1, 128), W2)

    y = _agg128(hs2, src, dst, zeros128)
    hs3 = _t1b(y, hs2, dinv,
               b2.reshape(1, 128), g2.reshape(1, 128), be2.reshape(1, 128))

    y3 = _agg128(hs3, src, dst, zeros128)
    out = _t2(y3, hs3, dinv, W3,
              b3.reshape(1, 64), in_w[128:192], in_b[128:192].reshape(1, 64),
              out_w, out_b.reshape(1, 64))
    return out[:N]
